# baseline (speedup 1.0000x reference)
import functools

import jax
import jax.numpy as jnp
from jax import lax
from jax.experimental import pallas as pl
from jax.experimental.pallas import tpu as pltpu
from jax.experimental.pallas import tpu_sc as plsc

N_ROIS = 32768
N_CLS = 80
NUM_WORKERS = 32
ROWS_PER_W = N_ROIS // NUM_WORKERS
IDX_MINOR = 128
N_GATHER_CHUNKS = ROWS_PER_W // IDX_MINOR

CE_BLOCK = 2048
CE_GRID = N_ROIS // CE_BLOCK


def _ce_body(x_ref, lab_ref, out_ref):
    i = pl.program_id(0)
    x = x_ref[...]
    lab = lab_ref[...]
    m = jnp.max(x, axis=-1, keepdims=True)
    s = jnp.sum(jnp.exp(x - m), axis=-1, keepdims=True)
    lse = m + jnp.log(s)
    cols = lax.broadcasted_iota(jnp.int32, x.shape, 1)
    sel = jnp.sum(jnp.where(cols == lab[:, None], x, 0.0), axis=-1,
                  keepdims=True)
    part = jnp.sum(lse - sel).reshape(1, 1)

    @pl.when(i == 0)
    def _():
        out_ref[...] = jnp.zeros((1, 1), jnp.float32)

    out_ref[...] += part


_ce_call = pl.pallas_call(
    _ce_body,
    grid=(CE_GRID,),
    in_specs=[
        pl.BlockSpec((CE_BLOCK, N_CLS + 1), lambda i: (i, 0)),
        pl.BlockSpec((CE_BLOCK,), lambda i: (i,)),
    ],
    out_specs=pl.BlockSpec((1, 1), lambda i: (0, 0)),
    out_shape=jax.ShapeDtypeStruct((1, 1), jnp.float32),
)


def _reg_body(table_hbm, lab_hbm, tgt_hbm, out_hbm,
              lab_v, idx_v, comp_v, tgt_v, acc_v, sem):
    wid = lax.axis_index("s") * 2 + lax.axis_index("c")
    base = wid * ROWS_PER_W

    pltpu.sync_copy(lab_hbm.at[pl.ds(base, ROWS_PER_W)], lab_v)
    for c in range(4):
        pltpu.sync_copy(
            tgt_hbm.at[pl.ds(c * N_ROIS + base, ROWS_PER_W)],
            tgt_v.at[pl.ds(c * ROWS_PER_W, ROWS_PER_W)])

    iota = lax.iota(jnp.int32, 16)

    for k in range(N_GATHER_CHUNKS):
        def build(j, carry, k=k):
            g = k * IDX_MINOR + j * 16
            lab = lab_v[pl.ds(g, 16)]
            lab = jnp.minimum(jnp.maximum(lab, 0), N_CLS - 1)
            fb = (base + g + iota) * (N_CLS * 4) + lab * 4
            for c in range(4):
                idx_v[c * N_GATHER_CHUNKS + k, pl.ds(j * 16, 16)] = fb + c
            return carry

        lax.fori_loop(0, IDX_MINOR // 16, build, 0)

    handles = []
    for c in range(4):
        for k in range(N_GATHER_CHUNKS):
            handles.append(pltpu.async_copy(
                table_hbm.at[idx_v.at[c * N_GATHER_CHUNKS + k]],
                comp_v.at[pl.ds(c * ROWS_PER_W + k * IDX_MINOR, IDX_MINOR)],
                sem))
    for h in handles:
        h.wait()

    def accum(j, acc):
        lab16 = lab_v[pl.ds(j * 16, 16)]
        w = jnp.where(lab16 < N_CLS, 1.0, 0.0).astype(jnp.float32)
        s = jnp.zeros((16,), jnp.float32)
        for c in range(4):
            o = c * ROWS_PER_W
            s = s + jnp.abs(comp_v[pl.ds(o + j * 16, 16)]
                            - tgt_v[pl.ds(o + j * 16, 16)])
        return acc + s * w

    acc = lax.fori_loop(0, ROWS_PER_W // 16, accum,
                        jnp.zeros((16,), jnp.float32))
    acc_v[...] = acc
    pltpu.sync_copy(acc_v, out_hbm.at[wid])


@functools.lru_cache(maxsize=1)
def _reg_call():
    return functools.partial(
        pl.kernel,
        out_type=jax.ShapeDtypeStruct((NUM_WORKERS, 16), jnp.float32),
        mesh=plsc.VectorSubcoreMesh(core_axis_name="c", subcore_axis_name="s"),
        scratch_types=[
            pltpu.VMEM((ROWS_PER_W,), jnp.int32),
            pltpu.VMEM((4 * N_GATHER_CHUNKS, IDX_MINOR), jnp.int32),
            pltpu.VMEM((ROWS_PER_W * 4,), jnp.float32),
            pltpu.VMEM((ROWS_PER_W * 4,), jnp.float32),
            pltpu.VMEM((16,), jnp.float32),
            pltpu.SemaphoreType.DMA,
        ],
    )(_reg_body)


def kernel(cls_preds, reg_preds, cls_labels, reg_targets):
    labels = cls_labels.astype(jnp.int32)
    table = reg_preds.reshape(N_ROIS * N_CLS * 4)
    tgt_flat = reg_targets.T.reshape(N_ROIS * 4)

    reg_parts = _reg_call()(table, labels, tgt_flat)
    cls_sum = _ce_call(cls_preds, labels)

    cls_loss = cls_sum[0, 0] / N_ROIS
    reg_loss = jnp.sum(reg_parts) / N_ROIS
    return cls_loss, reg_loss
